# baseline (speedup 1.0000x reference)
import functools

import jax
import jax.numpy as jnp
from jax import lax
from jax.experimental import pallas as pl
from jax.experimental.pallas import tpu as pltpu
from jax.experimental.pallas import tpu_sc as plsc

NUM_K = 1024
DIM = 256
TOTAL = 8 * 1024
BLK = 512


def _argmin_body(codes_ref, cb_ref, c2_ref, e2_ref, ids_ref):
    codes_blk = codes_ref[...]
    cb = cb_ref[...]
    dot = lax.dot_general(
        codes_blk, cb,
        dimension_numbers=(((1,), (1,)), ((), ())),
        preferred_element_type=jnp.float32,
    )
    d2 = (c2_ref[...] - 2.0 * dot) + e2_ref[...]
    dist = jnp.sqrt(jnp.maximum(d2, 0.0))
    minval = jnp.min(dist, axis=1, keepdims=True)
    iota = lax.broadcasted_iota(jnp.int32, dist.shape, 1)
    cand = jnp.where(dist == minval, iota, NUM_K)
    ids_ref[...] = jnp.min(cand, axis=1, keepdims=True)


def _compute_ids(codes2d, codebook, c2, e2):
    grid = TOTAL // BLK
    return pl.pallas_call(
        _argmin_body,
        grid=(grid,),
        in_specs=[
            pl.BlockSpec((BLK, DIM), lambda i: (i, 0)),
            pl.BlockSpec((NUM_K, DIM), lambda i: (0, 0)),
            pl.BlockSpec((BLK, 1), lambda i: (i, 0)),
            pl.BlockSpec((1, NUM_K), lambda i: (0, 0)),
        ],
        out_specs=pl.BlockSpec((BLK, 1), lambda i: (i, 0)),
        out_shape=jax.ShapeDtypeStruct((TOTAL, 1), jnp.int32),
    )(codes2d, codebook, c2, e2)


_NC, _NS = 2, 16
_NW = _NC * _NS
_B_PER_W = TOTAL // _NW
_IDX_CHUNK = 128
_CHUNKS = _B_PER_W // _IDX_CHUNK


def _gather_rows(codebook, ids2d):
    mesh = plsc.VectorSubcoreMesh(core_axis_name="c", subcore_axis_name="s")

    @functools.partial(
        pl.kernel,
        out_type=jax.ShapeDtypeStruct((TOTAL, DIM), jnp.float32),
        mesh=mesh,
        scratch_types=[
            pltpu.VMEM((_CHUNKS, _IDX_CHUNK), jnp.int32),
            pltpu.VMEM((_B_PER_W, DIM), jnp.float32),
            pltpu.SemaphoreType.DMA,
        ],
    )
    def k(table_hbm, idx_hbm, out_hbm, idx_v, rows_v, sem):
        wid = lax.axis_index("s") * _NC + lax.axis_index("c")
        base = wid * _B_PER_W
        pltpu.sync_copy(idx_hbm.at[pl.ds(wid * _CHUNKS, _CHUNKS)], idx_v)
        copies = [
            pltpu.async_copy(
                table_hbm.at[idx_v.at[j]],
                rows_v.at[pl.ds(j * _IDX_CHUNK, _IDX_CHUNK)],
                sem,
            )
            for j in range(_CHUNKS)
        ]
        for c in copies:
            c.wait()
        pltpu.sync_copy(rows_v, out_hbm.at[pl.ds(base, _B_PER_W)])

    return k(codebook, ids2d)


def kernel(codes, codebook):
    B, N, D = codes.shape
    codes2d = codes.reshape(B * N, D)
    c2 = jnp.sum(codes2d * codes2d, axis=-1, keepdims=True)
    e2 = jnp.sum(codebook * codebook, axis=-1)[None, :]
    ids = _compute_ids(codes2d, codebook, c2, e2)
    ids_flat = ids.reshape(TOTAL)
    quant = _gather_rows(codebook, ids_flat.reshape(_NW * _CHUNKS, _IDX_CHUNK))
    quant_id = ids_flat.reshape(B, N)
    return quant.reshape(B, N, D), quant_id


# speedup vs baseline: 1.0468x; 1.0468x over previous
import functools

import jax
import jax.numpy as jnp
from jax import lax
from jax.experimental import pallas as pl
from jax.experimental.pallas import tpu as pltpu
from jax.experimental.pallas import tpu_sc as plsc

NUM_K = 1024
DIM = 256
TOTAL = 8 * 1024
BLK = 2048


def _argmin_body(codes_ref, cb_ref, c2_ref, e2_ref, ids_ref):
    codes_blk = codes_ref[...]
    cb = cb_ref[...]
    dot = lax.dot_general(
        codes_blk, cb,
        dimension_numbers=(((1,), (1,)), ((), ())),
        preferred_element_type=jnp.float32,
    )
    d2 = (c2_ref[...] - 2.0 * dot) + e2_ref[...]
    dist = jnp.sqrt(jnp.maximum(d2, 0.0))
    minval = jnp.min(dist, axis=1, keepdims=True)
    iota = lax.broadcasted_iota(jnp.int32, dist.shape, 1)
    cand = jnp.where(dist == minval, iota, NUM_K)
    ids_ref[...] = jnp.min(cand, axis=1, keepdims=True)


def _compute_ids(codes2d, codebook, c2, e2):
    grid = TOTAL // BLK
    return pl.pallas_call(
        _argmin_body,
        grid=(grid,),
        in_specs=[
            pl.BlockSpec((BLK, DIM), lambda i: (i, 0)),
            pl.BlockSpec((NUM_K, DIM), lambda i: (0, 0)),
            pl.BlockSpec((BLK, 1), lambda i: (i, 0)),
            pl.BlockSpec((1, NUM_K), lambda i: (0, 0)),
        ],
        out_specs=pl.BlockSpec((BLK, 1), lambda i: (i, 0)),
        out_shape=jax.ShapeDtypeStruct((TOTAL, 1), jnp.int32),
    )(codes2d, codebook, c2, e2)


_NC, _NS = 2, 16
_NW = _NC * _NS
_B_PER_W = TOTAL // _NW
_IDX_CHUNK = 128
_CHUNKS = _B_PER_W // _IDX_CHUNK


def _gather_rows(codebook, ids2d):
    mesh = plsc.VectorSubcoreMesh(core_axis_name="c", subcore_axis_name="s")

    @functools.partial(
        pl.kernel,
        out_type=jax.ShapeDtypeStruct((TOTAL, DIM), jnp.float32),
        mesh=mesh,
        scratch_types=[
            pltpu.VMEM((_CHUNKS, _IDX_CHUNK), jnp.int32),
            pltpu.VMEM((_B_PER_W, DIM), jnp.float32),
            pltpu.SemaphoreType.DMA,
        ],
    )
    def k(table_hbm, idx_hbm, out_hbm, idx_v, rows_v, sem):
        wid = lax.axis_index("s") * _NC + lax.axis_index("c")
        base = wid * _B_PER_W
        pltpu.sync_copy(idx_hbm.at[pl.ds(wid * _CHUNKS, _CHUNKS)], idx_v)
        copies = [
            pltpu.async_copy(
                table_hbm.at[idx_v.at[j]],
                rows_v.at[pl.ds(j * _IDX_CHUNK, _IDX_CHUNK)],
                sem,
            )
            for j in range(_CHUNKS)
        ]
        for c in copies:
            c.wait()
        pltpu.sync_copy(rows_v, out_hbm.at[pl.ds(base, _B_PER_W)])

    return k(codebook, ids2d)


def kernel(codes, codebook):
    B, N, D = codes.shape
    codes2d = codes.reshape(B * N, D)
    c2 = jnp.sum(codes2d * codes2d, axis=-1, keepdims=True)
    e2 = jnp.sum(codebook * codebook, axis=-1)[None, :]
    ids = _compute_ids(codes2d, codebook, c2, e2)
    ids_flat = ids.reshape(TOTAL)
    quant = _gather_rows(codebook, ids_flat.reshape(_NW * _CHUNKS, _IDX_CHUNK))
    quant_id = ids_flat.reshape(B, N)
    return quant.reshape(B, N, D), quant_id


# speedup vs baseline: 1.0961x; 1.0471x over previous
import functools

import jax
import jax.numpy as jnp
from jax import lax
from jax.experimental import pallas as pl
from jax.experimental.pallas import tpu as pltpu
from jax.experimental.pallas import tpu_sc as plsc

NUM_K = 1024
DIM = 256
TOTAL = 8 * 1024
BLK = 2048


def _argmin_body(codes_ref, cb_ref, c2_ref, e2_ref, iota_ref, ids_ref):
    codes_blk = codes_ref[...] * -2.0
    cb = cb_ref[...]
    dot2 = lax.dot_general(
        codes_blk, cb,
        dimension_numbers=(((1,), (1,)), ((), ())),
        preferred_element_type=jnp.float32,
    )
    d2 = (c2_ref[...] + dot2) + e2_ref[...]
    dist = jnp.sqrt(jnp.maximum(d2, 0.0))
    minval = jnp.min(dist, axis=1, keepdims=True)
    cand = jnp.where(dist == minval, iota_ref[...], float(NUM_K))
    ids_ref[...] = jnp.min(cand, axis=1, keepdims=True)


def _compute_ids(codes2d, codebook, c2, e2):
    grid = TOTAL // BLK
    iota = jnp.arange(NUM_K, dtype=jnp.float32)[None, :]
    idsf = pl.pallas_call(
        _argmin_body,
        grid=(grid,),
        in_specs=[
            pl.BlockSpec((BLK, DIM), lambda i: (i, 0)),
            pl.BlockSpec((NUM_K, DIM), lambda i: (0, 0)),
            pl.BlockSpec((BLK, 1), lambda i: (i, 0)),
            pl.BlockSpec((1, NUM_K), lambda i: (0, 0)),
            pl.BlockSpec((1, NUM_K), lambda i: (0, 0)),
        ],
        out_specs=pl.BlockSpec((BLK, 1), lambda i: (i, 0)),
        out_shape=jax.ShapeDtypeStruct((TOTAL, 1), jnp.float32),
    )(codes2d, codebook, c2, e2, iota)
    return idsf.astype(jnp.int32)


_NC, _NS = 2, 16
_NW = _NC * _NS
_B_PER_W = TOTAL // _NW
_IDX_CHUNK = 128
_CHUNKS = _B_PER_W // _IDX_CHUNK


def _gather_rows(codebook, ids2d):
    mesh = plsc.VectorSubcoreMesh(core_axis_name="c", subcore_axis_name="s")

    @functools.partial(
        pl.kernel,
        out_type=jax.ShapeDtypeStruct((TOTAL, DIM), jnp.float32),
        mesh=mesh,
        scratch_types=[
            pltpu.VMEM((_CHUNKS, _IDX_CHUNK), jnp.int32),
            pltpu.VMEM((_B_PER_W, DIM), jnp.float32),
            pltpu.SemaphoreType.DMA,
        ],
    )
    def k(table_hbm, idx_hbm, out_hbm, idx_v, rows_v, sem):
        wid = lax.axis_index("s") * _NC + lax.axis_index("c")
        base = wid * _B_PER_W
        pltpu.sync_copy(idx_hbm.at[pl.ds(wid * _CHUNKS, _CHUNKS)], idx_v)
        copies = [
            pltpu.async_copy(
                table_hbm.at[idx_v.at[j]],
                rows_v.at[pl.ds(j * _IDX_CHUNK, _IDX_CHUNK)],
                sem,
            )
            for j in range(_CHUNKS)
        ]
        for c in copies:
            c.wait()
        pltpu.sync_copy(rows_v, out_hbm.at[pl.ds(base, _B_PER_W)])

    return k(codebook, ids2d)


def kernel(codes, codebook):
    B, N, D = codes.shape
    codes2d = codes.reshape(B * N, D)
    c2 = jnp.sum(codes2d * codes2d, axis=-1, keepdims=True)
    e2 = jnp.sum(codebook * codebook, axis=-1)[None, :]
    ids = _compute_ids(codes2d, codebook, c2, e2)
    ids_flat = ids.reshape(TOTAL)
    quant = _gather_rows(codebook, ids_flat.reshape(_NW * _CHUNKS, _IDX_CHUNK))
    quant_id = ids_flat.reshape(B, N)
    return quant.reshape(B, N, D), quant_id


# speedup vs baseline: 1.2310x; 1.1231x over previous
import functools

import jax
import jax.numpy as jnp
from jax import lax
from jax.experimental import pallas as pl
from jax.experimental.pallas import tpu as pltpu
from jax.experimental.pallas import tpu_sc as plsc

NUM_K = 1024
DIM = 256
TOTAL = 8 * 1024
BLK = 2048


def _argmin_body(codes_ref, cb_ref, e2_ref, iota_ref, ids_ref):
    codes = codes_ref[...]
    c2 = jnp.sum(codes * codes, axis=1, keepdims=True)
    cb = cb_ref[...]
    dot2 = lax.dot_general(
        codes * -2.0, cb,
        dimension_numbers=(((1,), (1,)), ((), ())),
        preferred_element_type=jnp.float32,
    )
    d2 = (c2 + dot2) + e2_ref[...]
    dist = jnp.sqrt(jnp.maximum(d2, 0.0))
    minval = jnp.min(dist, axis=1, keepdims=True)
    cand = jnp.where(dist == minval, iota_ref[...], float(NUM_K))
    ids_ref[...] = jnp.min(cand, axis=1, keepdims=True).astype(jnp.int32)


def _compute_ids(codes2d, codebook, e2):
    grid = TOTAL // BLK
    iota = jnp.arange(NUM_K, dtype=jnp.float32)[None, :]
    return pl.pallas_call(
        _argmin_body,
        grid=(grid,),
        in_specs=[
            pl.BlockSpec((BLK, DIM), lambda i: (i, 0)),
            pl.BlockSpec((NUM_K, DIM), lambda i: (0, 0)),
            pl.BlockSpec((1, NUM_K), lambda i: (0, 0)),
            pl.BlockSpec((1, NUM_K), lambda i: (0, 0)),
        ],
        out_specs=pl.BlockSpec((BLK, 1), lambda i: (i, 0)),
        out_shape=jax.ShapeDtypeStruct((TOTAL, 1), jnp.int32),
    )(codes2d, codebook, e2, iota)


_NC, _NS = 2, 16
_NW = _NC * _NS
_B_PER_W = TOTAL // _NW
_IDX_CHUNK = 128
_CHUNKS = _B_PER_W // _IDX_CHUNK


def _gather_rows(codebook, ids2d):
    mesh = plsc.VectorSubcoreMesh(core_axis_name="c", subcore_axis_name="s")

    @functools.partial(
        pl.kernel,
        out_type=jax.ShapeDtypeStruct((TOTAL, DIM), jnp.float32),
        mesh=mesh,
        scratch_types=[
            pltpu.VMEM((_CHUNKS, _IDX_CHUNK), jnp.int32),
            pltpu.VMEM((_B_PER_W, DIM), jnp.float32),
            pltpu.SemaphoreType.DMA,
        ],
    )
    def k(table_hbm, idx_hbm, out_hbm, idx_v, rows_v, sem):
        wid = lax.axis_index("s") * _NC + lax.axis_index("c")
        base = wid * _B_PER_W
        pltpu.sync_copy(idx_hbm.at[pl.ds(wid * _CHUNKS, _CHUNKS)], idx_v)
        copies = [
            pltpu.async_copy(
                table_hbm.at[idx_v.at[j]],
                rows_v.at[pl.ds(j * _IDX_CHUNK, _IDX_CHUNK)],
                sem,
            )
            for j in range(_CHUNKS)
        ]
        for c in copies:
            c.wait()
        pltpu.sync_copy(rows_v, out_hbm.at[pl.ds(base, _B_PER_W)])

    return k(codebook, ids2d)


def kernel(codes, codebook):
    B, N, D = codes.shape
    codes2d = codes.reshape(B * N, D)
    e2 = jnp.sum(codebook * codebook, axis=-1)[None, :]
    ids = _compute_ids(codes2d, codebook, e2)
    ids_flat = ids.reshape(TOTAL)
    quant = _gather_rows(codebook, ids_flat.reshape(_NW * _CHUNKS, _IDX_CHUNK))
    quant_id = ids_flat.reshape(B, N)
    return quant.reshape(B, N, D), quant_id
